# baseline (speedup 1.0000x reference)
import functools
import math

import numpy as np
import jax
import jax.numpy as jnp
from jax import lax
from jax.experimental import pallas as pl
from jax.experimental.pallas import tpu as pltpu
from jax.experimental.pallas import tpu_sc as plsc

NHEAD = 8
WSZ = 5
_F32 = jnp.float32


def _sine_pos_encoding_np(C, H, W):
    pe = np.zeros((C, H, W), dtype=np.float32)
    yy = np.tile(np.arange(H, dtype=np.float32)[:, None], (1, W))
    xx = np.tile(np.arange(W, dtype=np.float32)[None, :], (H, 1))
    div = np.exp(np.arange(0, C // 2, 2).astype(np.float32) * (-math.log(10000.0) / (C // 2)))
    d = div[:, None, None]
    pe[0::4] = np.sin(xx[None] * d)
    pe[1::4] = np.cos(xx[None] * d)
    pe[2::4] = np.sin(yy[None] * d)
    pe[3::4] = np.cos(yy[None] * d)
    return pe


def _window_valid_np(hh, ww, wsz):
    L = hh * ww
    r = np.arange(L) // ww
    c = np.arange(L) % ww
    off = np.arange(wsz) - wsz // 2
    dr = np.repeat(off, wsz)
    dc = np.tile(off, wsz)
    rr = r[:, None] + dr[None, :]
    cc = c[:, None] + dc[None, :]
    return ((rr >= 0) & (rr < hh) & (cc >= 0) & (cc < ww)).astype(np.float32)


_M = 1536


def _sc_compact(tok_pad, L):
    npad = tok_pad.shape[1]
    mesh = plsc.VectorSubcoreMesh(core_axis_name="c", subcore_axis_name="s")

    @functools.partial(
        pl.kernel,
        out_type=jax.ShapeDtypeStruct((2, _M), jnp.int32),
        mesh=mesh,
        scratch_types=[
            pltpu.VMEM((npad,), jnp.int32),
            pltpu.VMEM((L,), _F32),
            pltpu.VMEM((_M,), jnp.int32),
        ],
        compiler_params=pltpu.CompilerParams(needs_layout_passes=False),
    )
    def build(tok_hbm, kidx_hbm, tok_v, mask_v, kidx_v):
        wid = lax.axis_index("s") * 2 + lax.axis_index("c")

        @pl.when(wid < 2)
        def _():
            pltpu.sync_copy(tok_hbm.at[wid], tok_v)
            zeros = jnp.zeros((16,), _F32)
            ones = jnp.ones((16,), _F32)
            ineg = jnp.full((16,), -1, jnp.int32)
            lanes = lax.iota(jnp.int32, 16)

            def init_body(i, carry):
                mask_v[pl.ds(i * 16, 16)] = zeros
                return carry

            lax.fori_loop(0, L // 16, init_body, 0)

            def scat_body(i, carry):
                idx = tok_v[pl.ds(i * 16, 16)]
                plsc.store_scatter(mask_v, [idx], ones)
                return carry

            lax.fori_loop(0, npad // 16, scat_body, 0)

            def kinit_body(i, carry):
                kidx_v[pl.ds(i * 16, 16)] = ineg
                return carry

            lax.fori_loop(0, _M // 16, kinit_body, 0)

            def compact_body(i, base):
                m = mask_v[pl.ds(i * 16, 16)]
                hit = m > 0.5
                c = plsc.cumsum(m)
                pos = (base + c - 1.0).astype(jnp.int32)
                pos = jnp.where(hit, pos, 0)
                tid = lanes + i * 16
                plsc.store_scatter(kidx_v, [pos], tid, mask=hit)
                return base + jnp.sum(m)

            lax.fori_loop(0, L // 16, compact_body, jnp.float32(0.0))
            pltpu.sync_copy(kidx_v, kidx_hbm.at[wid])

    return build(tok_pad)


def _ffn_block(o, w1, w2):
    mu = jnp.mean(o, axis=-1, keepdims=True)
    var = jnp.mean((o - mu) * (o - mu), axis=-1, keepdims=True)
    ln = (o - mu) / jnp.sqrt(var + 1e-6)
    h = jnp.maximum(jnp.dot(ln, w1, preferred_element_type=_F32), 0.0)
    return o + jnp.dot(h, w2, preferred_element_type=_F32)


def _self_body(x_ref, pe_ref, ki_ref, wq_ref, wk_ref, wv_ref, wo_ref,
               w1_ref, w2_ref, o_ref, q_s, k_s, v_s, *, L, C, hd, tq):
    x = x_ref[0] + pe_ref[...]
    scale = 1.0 / math.sqrt(hd)
    q_s[...] = (jnp.dot(x, wq_ref[...], preferred_element_type=_F32)
                * scale).astype(jnp.bfloat16)
    kcol = jnp.transpose(ki_ref[0])
    oh = (kcol == lax.broadcasted_iota(jnp.int32, (_M, L), 1)
          ).astype(jnp.bfloat16)
    xg = jnp.dot(oh, x.astype(jnp.bfloat16), preferred_element_type=_F32)
    k_s[...] = jnp.dot(xg, wk_ref[...],
                       preferred_element_type=_F32).astype(jnp.bfloat16)
    v_s[...] = jnp.dot(xg, wv_ref[...],
                       preferred_element_type=_F32).astype(jnp.bfloat16)
    npad = jnp.sum((ki_ref[0] == -1).astype(_F32), axis=-1, keepdims=True)
    wo = wo_ref[...]
    w1 = w1_ref[...]
    w2 = w2_ref[...]
    ones_col = jnp.ones((_M, 1), jnp.bfloat16)
    for t in range(L // tq):
        sl = slice(t * tq, (t + 1) * tq)
        msg_parts = []
        for h in range(NHEAD):
            hs = slice(h * hd, (h + 1) * hd)
            lg = lax.dot_general(q_s[sl, hs], k_s[:, hs],
                                 (((1,), (1,)), ((), ())),
                                 preferred_element_type=_F32)
            p = jnp.exp(lg).astype(jnp.bfloat16)
            s = jnp.dot(p, ones_col, preferred_element_type=_F32) - npad
            pv = jnp.dot(p, v_s[:, hs], preferred_element_type=_F32)
            msg_parts.append(pv * (1.0 / s))
        msg = jnp.concatenate(msg_parts, axis=-1)
        o = (x_ref[0, sl, :] + pe_ref[sl, :]
             + jnp.dot(msg, wo, preferred_element_type=_F32))
        o_ref[0, sl, :] = _ffn_block(o, w1, w2)


def _self_layer(x_raw, pe, kidx, wq, wk, wv, wo, w1, w2):
    _, L, C = x_raw.shape
    hd = C // NHEAD
    body = functools.partial(_self_body, L=L, C=C, hd=hd, tq=512)
    full2 = lambda i: (0, 0)
    return pl.pallas_call(
        body,
        grid=(2,),
        in_specs=[
            pl.BlockSpec((1, L, C), lambda i: (i, 0, 0)),
            pl.BlockSpec((L, C), full2),
            pl.BlockSpec((1, 1, _M), lambda i: (i, 0, 0)),
            pl.BlockSpec((C, C), full2),
            pl.BlockSpec((C, C), full2),
            pl.BlockSpec((C, C), full2),
            pl.BlockSpec((C, C), full2),
            pl.BlockSpec((C, 2 * C), full2),
            pl.BlockSpec((2 * C, C), full2),
        ],
        out_specs=pl.BlockSpec((1, L, C), lambda i: (i, 0, 0)),
        out_shape=jax.ShapeDtypeStruct((2, L, C), _F32),
        scratch_shapes=[
            pltpu.VMEM((L, C), jnp.bfloat16),
            pltpu.VMEM((_M, C), jnp.bfloat16),
            pltpu.VMEM((_M, C), jnp.bfloat16),
        ],
    )(x_raw, pe, kidx, wq, wk, wv, wo, w1, w2)


_HALO = 136


def _cross_body(xq_ref, xkv_ref, wm_ref, wq_ref, wk_ref, wv_ref, wo_ref,
                w1_ref, w2_ref, o_ref, kpad_ref, vpad_ref,
                *, L, C, hd, shifts, tq):
    xq = xq_ref[0]
    xkv = xkv_ref[0]
    bf16 = jnp.bfloat16
    kpad_ref[:_HALO, :] = jnp.zeros((_HALO, C), bf16)
    kpad_ref[_HALO + L:, :] = jnp.zeros((_HALO, C), bf16)
    vpad_ref[:_HALO, :] = jnp.zeros((_HALO, C), bf16)
    vpad_ref[_HALO + L:, :] = jnp.zeros((_HALO, C), bf16)
    kpad_ref[_HALO:_HALO + L, :] = jnp.dot(
        xkv, wk_ref[...], preferred_element_type=_F32).astype(bf16)
    vpad_ref[_HALO:_HALO + L, :] = jnp.dot(
        xkv, wv_ref[...], preferred_element_type=_F32).astype(bf16)
    scale = 1.0 / math.sqrt(hd)
    di = lax.broadcasted_iota(jnp.int32, (C, NHEAD), 0)
    hi = lax.broadcasted_iota(jnp.int32, (C, NHEAD), 1)
    e = (di // hd == hi).astype(bf16)
    et32 = (lax.broadcasted_iota(jnp.int32, (NHEAD, C), 1) // hd ==
            lax.broadcasted_iota(jnp.int32, (NHEAD, C), 0)).astype(_F32)
    et = et32.astype(bf16)
    wq = wq_ref[...]
    wo = wo_ref[...]
    w1 = w1_ref[...]
    w2 = w2_ref[...]

    for t in range(L // tq):
        sl = slice(t * tq, (t + 1) * tq)
        xq_t = xq[sl]
        q_t = (jnp.dot(xq_t, wq, preferred_element_type=_F32)
               * scale).astype(bf16)
        lgts = []
        for w, s in enumerate(shifts):
            ks = kpad_ref[_HALO + t * tq + s:_HALO + t * tq + s + tq, :]
            lg = jnp.dot(q_t * ks, e, preferred_element_type=_F32)
            valid = wm_ref[sl, w:w + 1]
            lgts.append(jnp.where(valid > 0.5, lg, -1e9))
        mx = lgts[0]
        for lg in lgts[1:]:
            mx = jnp.maximum(mx, lg)
        ps = [jnp.exp(lg - mx) for lg in lgts]
        denom = ps[0]
        for p in ps[1:]:
            denom = denom + p
        msg = jnp.zeros((tq, C), _F32)
        for w, s in enumerate(shifts):
            vs = vpad_ref[_HALO + t * tq + s:_HALO + t * tq + s + tq, :]
            pexp = jnp.dot(ps[w].astype(bf16), et,
                           preferred_element_type=_F32).astype(bf16)
            msg = msg + (pexp * vs).astype(_F32)
        msg = msg * jnp.dot(1.0 / denom, et32, preferred_element_type=_F32)
        o = xq_t + jnp.dot(msg, wo, preferred_element_type=_F32)
        o_ref[0, sl, :] = _ffn_block(o, w1, w2)


def _cross_layer(xq, wmask, wq, wk, wv, wo, w1, w2, shifts):
    _, L, C = xq.shape
    hd = C // NHEAD
    body = functools.partial(_cross_body, L=L, C=C, hd=hd, shifts=shifts,
                             tq=512)
    full2 = lambda i: (0, 0)
    return pl.pallas_call(
        body,
        grid=(2,),
        in_specs=[
            pl.BlockSpec((1, L, C), lambda i: (i, 0, 0)),
            pl.BlockSpec((1, L, C), lambda i: (1 - i, 0, 0)),
            pl.BlockSpec((L, WSZ * WSZ), full2),
            pl.BlockSpec((C, C), full2),
            pl.BlockSpec((C, C), full2),
            pl.BlockSpec((C, C), full2),
            pl.BlockSpec((C, C), full2),
            pl.BlockSpec((C, 2 * C), full2),
            pl.BlockSpec((2 * C, C), full2),
        ],
        out_specs=pl.BlockSpec((1, L, C), lambda i: (i, 0, 0)),
        out_shape=jax.ShapeDtypeStruct((2, L, C), _F32),
        scratch_shapes=[
            pltpu.VMEM((2 * _HALO + L, C), jnp.bfloat16),
            pltpu.VMEM((2 * _HALO + L, C), jnp.bfloat16),
        ],
    )(xq, xq, wmask, wq, wk, wv, wo, w1, w2)


def kernel(cnn_desc0, cnn_desc1, mkpts0_c, mkpts1_c, m_bids, image0, image1,
           Wq, Wk, Wv, Wo, W1, W2):
    B, C, hh, ww = cnn_desc0.shape
    L = hh * ww
    scale = image0.shape[2] // hh
    hd = C // NHEAD

    pe = jnp.asarray(_sine_pos_encoding_np(C, hh, ww).reshape(C, L).T)
    wmask = jnp.asarray(_window_valid_np(hh, ww, WSZ))
    off = np.arange(WSZ) - WSZ // 2
    shifts = [int(dr) * ww + int(dc) for dr in off for dc in off]

    x_raw = jnp.stack([
        cnn_desc0.reshape(C, L).T,
        cnn_desc1.reshape(C, L).T,
    ])

    tok0 = (mkpts0_c[:, 1] // scale) * ww + (mkpts0_c[:, 0] // scale)
    tok1 = (mkpts1_c[:, 1] // scale) * ww + (mkpts1_c[:, 0] // scale)
    tok = jnp.stack([tok0, tok1]).astype(jnp.int32)
    pad = (-tok.shape[1]) % 16
    if pad:
        tok = jnp.concatenate([tok, tok[:, :pad]], axis=1)
    kidx = _sc_compact(tok, L).reshape(2, 1, _M)

    xs = _self_layer(x_raw, pe, kidx,
                     Wq[0], Wk[0], Wv[0], Wo[0], W1[0], W2[0])
    xc = _cross_layer(xs, wmask, Wq[1], Wk[1], Wv[1], Wo[1], W1[1], W2[1],
                      shifts)
    return xc[0][None], xc[1][None]


# speedup vs baseline: 1.0850x; 1.0850x over previous
import functools
import math

import numpy as np
import jax
import jax.numpy as jnp
from jax import lax
from jax.experimental import pallas as pl
from jax.experimental.pallas import tpu as pltpu
from jax.experimental.pallas import tpu_sc as plsc

NHEAD = 8
WSZ = 5
_F32 = jnp.float32


def _sine_pos_encoding_np(C, H, W):
    pe = np.zeros((C, H, W), dtype=np.float32)
    yy = np.tile(np.arange(H, dtype=np.float32)[:, None], (1, W))
    xx = np.tile(np.arange(W, dtype=np.float32)[None, :], (H, 1))
    div = np.exp(np.arange(0, C // 2, 2).astype(np.float32) * (-math.log(10000.0) / (C // 2)))
    d = div[:, None, None]
    pe[0::4] = np.sin(xx[None] * d)
    pe[1::4] = np.cos(xx[None] * d)
    pe[2::4] = np.sin(yy[None] * d)
    pe[3::4] = np.cos(yy[None] * d)
    return pe


def _window_valid_np(hh, ww, wsz):
    L = hh * ww
    r = np.arange(L) // ww
    c = np.arange(L) % ww
    off = np.arange(wsz) - wsz // 2
    dr = np.repeat(off, wsz)
    dc = np.tile(off, wsz)
    rr = r[:, None] + dr[None, :]
    cc = c[:, None] + dc[None, :]
    return ((rr >= 0) & (rr < hh) & (cc >= 0) & (cc < ww)).astype(np.float32)


_M = 1536


def _sc_compact(tok_pad, L):
    npad = tok_pad.shape[1]
    mesh = plsc.VectorSubcoreMesh(core_axis_name="c", subcore_axis_name="s")

    @functools.partial(
        pl.kernel,
        out_type=jax.ShapeDtypeStruct((2, _M), jnp.int32),
        mesh=mesh,
        scratch_types=[
            pltpu.VMEM((npad,), jnp.int32),
            pltpu.VMEM((L,), _F32),
            pltpu.VMEM((_M,), jnp.int32),
        ],
        compiler_params=pltpu.CompilerParams(needs_layout_passes=False),
    )
    def build(tok_hbm, kidx_hbm, tok_v, mask_v, kidx_v):
        wid = lax.axis_index("s") * 2 + lax.axis_index("c")

        @pl.when(wid < 2)
        def _():
            pltpu.sync_copy(tok_hbm.at[wid], tok_v)
            zeros = jnp.zeros((16,), _F32)
            ones = jnp.ones((16,), _F32)
            ineg = jnp.full((16,), -1, jnp.int32)
            lanes = lax.iota(jnp.int32, 16)

            def init_body(i, carry):
                mask_v[pl.ds(i * 16, 16)] = zeros
                return carry

            lax.fori_loop(0, L // 16, init_body, 0)

            def scat_body(i, carry):
                idx = tok_v[pl.ds(i * 16, 16)]
                plsc.store_scatter(mask_v, [idx], ones)
                return carry

            lax.fori_loop(0, npad // 16, scat_body, 0)

            def kinit_body(i, carry):
                kidx_v[pl.ds(i * 16, 16)] = ineg
                return carry

            lax.fori_loop(0, _M // 16, kinit_body, 0)

            def compact_body(i, base):
                m = mask_v[pl.ds(i * 16, 16)]
                hit = m > 0.5
                c = plsc.cumsum(m)
                pos = (base + c - 1.0).astype(jnp.int32)
                pos = jnp.where(hit, pos, 0)
                tid = lanes + i * 16
                plsc.store_scatter(kidx_v, [pos], tid, mask=hit)
                return base + jnp.sum(m)

            lax.fori_loop(0, L // 16, compact_body, jnp.float32(0.0))
            pltpu.sync_copy(kidx_v, kidx_hbm.at[wid])

    return build(tok_pad)


def _ffn_block(o, w1, w2):
    mu = jnp.mean(o, axis=-1, keepdims=True)
    var = jnp.mean((o - mu) * (o - mu), axis=-1, keepdims=True)
    ln = (o - mu) / jnp.sqrt(var + 1e-6)
    h = jnp.maximum(jnp.dot(ln, w1, preferred_element_type=_F32), 0.0)
    return o + jnp.dot(h, w2, preferred_element_type=_F32)


def _self_body(x_ref, pe_ref, ki_ref, wq_ref, wk_ref, wv_ref, wo_ref,
               w1_ref, w2_ref, o_ref, q_s, k_s, v_s, *, L, C, hd, tq):
    x = x_ref[0] + pe_ref[...]
    scale = 1.0 / math.sqrt(hd)
    q_s[...] = (jnp.dot(x, wq_ref[...], preferred_element_type=_F32)
                * scale).astype(jnp.bfloat16)
    kcol = jnp.transpose(ki_ref[0])
    oh = (kcol == lax.broadcasted_iota(jnp.int32, (_M, L), 1)
          ).astype(jnp.bfloat16)
    xg = jnp.dot(oh, x.astype(jnp.bfloat16), preferred_element_type=_F32)
    k_s[...] = jnp.dot(xg, wk_ref[...],
                       preferred_element_type=_F32).astype(jnp.bfloat16)
    v_s[...] = jnp.dot(xg, wv_ref[...],
                       preferred_element_type=_F32).astype(jnp.bfloat16)
    npad = jnp.sum((ki_ref[0] == -1).astype(_F32), axis=-1, keepdims=True)
    wo = wo_ref[...]
    w1 = w1_ref[...]
    w2 = w2_ref[...]
    for t in range(L // tq):
        sl = slice(t * tq, (t + 1) * tq)
        msg_parts = []
        for h in range(NHEAD):
            hs = slice(h * hd, (h + 1) * hd)
            lg = lax.dot_general(q_s[sl, hs], k_s[:, hs],
                                 (((1,), (1,)), ((), ())),
                                 preferred_element_type=_F32)
            p32 = jnp.exp(lg)
            s = jnp.sum(p32, axis=-1, keepdims=True) - npad
            pv = jnp.dot(p32.astype(jnp.bfloat16), v_s[:, hs],
                         preferred_element_type=_F32)
            msg_parts.append(pv * (1.0 / s))
        msg = jnp.concatenate(msg_parts, axis=-1)
        o = (x_ref[0, sl, :] + pe_ref[sl, :]
             + jnp.dot(msg, wo, preferred_element_type=_F32))
        o_ref[0, sl, :] = _ffn_block(o, w1, w2)


def _self_layer(x_raw, pe, kidx, wq, wk, wv, wo, w1, w2):
    _, L, C = x_raw.shape
    hd = C // NHEAD
    body = functools.partial(_self_body, L=L, C=C, hd=hd, tq=1024)
    full2 = lambda i: (0, 0)
    return pl.pallas_call(
        body,
        grid=(2,),
        in_specs=[
            pl.BlockSpec((1, L, C), lambda i: (i, 0, 0)),
            pl.BlockSpec((L, C), full2),
            pl.BlockSpec((1, 1, _M), lambda i: (i, 0, 0)),
            pl.BlockSpec((C, C), full2),
            pl.BlockSpec((C, C), full2),
            pl.BlockSpec((C, C), full2),
            pl.BlockSpec((C, C), full2),
            pl.BlockSpec((C, 2 * C), full2),
            pl.BlockSpec((2 * C, C), full2),
        ],
        out_specs=pl.BlockSpec((1, L, C), lambda i: (i, 0, 0)),
        out_shape=jax.ShapeDtypeStruct((2, L, C), _F32),
        scratch_shapes=[
            pltpu.VMEM((L, C), jnp.bfloat16),
            pltpu.VMEM((_M, C), jnp.bfloat16),
            pltpu.VMEM((_M, C), jnp.bfloat16),
        ],
    )(x_raw, pe, kidx, wq, wk, wv, wo, w1, w2)


_HALO = 136


def _cross_body(xq_ref, xkv_ref, wm_ref, wq_ref, wk_ref, wv_ref, wo_ref,
                w1_ref, w2_ref, o_ref, kpad_ref, vpad_ref,
                *, L, C, hd, shifts, tq):
    xq = xq_ref[0]
    xkv = xkv_ref[0]
    bf16 = jnp.bfloat16
    kpad_ref[:_HALO, :] = jnp.zeros((_HALO, C), bf16)
    kpad_ref[_HALO + L:, :] = jnp.zeros((_HALO, C), bf16)
    vpad_ref[:_HALO, :] = jnp.zeros((_HALO, C), bf16)
    vpad_ref[_HALO + L:, :] = jnp.zeros((_HALO, C), bf16)
    kpad_ref[_HALO:_HALO + L, :] = jnp.dot(
        xkv, wk_ref[...], preferred_element_type=_F32).astype(bf16)
    vpad_ref[_HALO:_HALO + L, :] = jnp.dot(
        xkv, wv_ref[...], preferred_element_type=_F32).astype(bf16)
    scale = 1.0 / math.sqrt(hd)
    di = lax.broadcasted_iota(jnp.int32, (C, NHEAD), 0)
    hi = lax.broadcasted_iota(jnp.int32, (C, NHEAD), 1)
    e = (di // hd == hi).astype(bf16)
    et32 = (lax.broadcasted_iota(jnp.int32, (NHEAD, C), 1) // hd ==
            lax.broadcasted_iota(jnp.int32, (NHEAD, C), 0)).astype(_F32)
    et = et32.astype(bf16)
    wq = wq_ref[...]
    wo = wo_ref[...]
    w1 = w1_ref[...]
    w2 = w2_ref[...]

    for t in range(L // tq):
        sl = slice(t * tq, (t + 1) * tq)
        xq_t = xq[sl]
        q_t = (jnp.dot(xq_t, wq, preferred_element_type=_F32)
               * scale).astype(bf16)
        lgts = []
        for w, s in enumerate(shifts):
            ks = kpad_ref[_HALO + t * tq + s:_HALO + t * tq + s + tq, :]
            lg = jnp.dot(q_t * ks, e, preferred_element_type=_F32)
            valid = wm_ref[sl, w:w + 1]
            lgts.append(jnp.where(valid > 0.5, lg, -1e9))
        mx = lgts[0]
        for lg in lgts[1:]:
            mx = jnp.maximum(mx, lg)
        ps = [jnp.exp(lg - mx) for lg in lgts]
        denom = ps[0]
        for p in ps[1:]:
            denom = denom + p
        msg = jnp.zeros((tq, C), _F32)
        for w, s in enumerate(shifts):
            vs = vpad_ref[_HALO + t * tq + s:_HALO + t * tq + s + tq, :]
            pexp = jnp.dot(ps[w].astype(bf16), et,
                           preferred_element_type=_F32).astype(bf16)
            msg = msg + (pexp * vs).astype(_F32)
        msg = msg * jnp.dot(1.0 / denom, et32, preferred_element_type=_F32)
        o = xq_t + jnp.dot(msg, wo, preferred_element_type=_F32)
        o_ref[0, sl, :] = _ffn_block(o, w1, w2)


def _cross_layer(xq, wmask, wq, wk, wv, wo, w1, w2, shifts):
    _, L, C = xq.shape
    hd = C // NHEAD
    body = functools.partial(_cross_body, L=L, C=C, hd=hd, shifts=shifts,
                             tq=1024)
    full2 = lambda i: (0, 0)
    return pl.pallas_call(
        body,
        grid=(2,),
        in_specs=[
            pl.BlockSpec((1, L, C), lambda i: (i, 0, 0)),
            pl.BlockSpec((1, L, C), lambda i: (1 - i, 0, 0)),
            pl.BlockSpec((L, WSZ * WSZ), full2),
            pl.BlockSpec((C, C), full2),
            pl.BlockSpec((C, C), full2),
            pl.BlockSpec((C, C), full2),
            pl.BlockSpec((C, C), full2),
            pl.BlockSpec((C, 2 * C), full2),
            pl.BlockSpec((2 * C, C), full2),
        ],
        out_specs=pl.BlockSpec((1, L, C), lambda i: (i, 0, 0)),
        out_shape=jax.ShapeDtypeStruct((2, L, C), _F32),
        scratch_shapes=[
            pltpu.VMEM((2 * _HALO + L, C), jnp.bfloat16),
            pltpu.VMEM((2 * _HALO + L, C), jnp.bfloat16),
        ],
    )(xq, xq, wmask, wq, wk, wv, wo, w1, w2)


def kernel(cnn_desc0, cnn_desc1, mkpts0_c, mkpts1_c, m_bids, image0, image1,
           Wq, Wk, Wv, Wo, W1, W2):
    B, C, hh, ww = cnn_desc0.shape
    L = hh * ww
    scale = image0.shape[2] // hh
    hd = C // NHEAD

    pe = jnp.asarray(_sine_pos_encoding_np(C, hh, ww).reshape(C, L).T)
    wmask = jnp.asarray(_window_valid_np(hh, ww, WSZ))
    off = np.arange(WSZ) - WSZ // 2
    shifts = [int(dr) * ww + int(dc) for dr in off for dc in off]

    x_raw = jnp.stack([
        cnn_desc0.reshape(C, L).T,
        cnn_desc1.reshape(C, L).T,
    ])

    tok0 = (mkpts0_c[:, 1] // scale) * ww + (mkpts0_c[:, 0] // scale)
    tok1 = (mkpts1_c[:, 1] // scale) * ww + (mkpts1_c[:, 0] // scale)
    tok = jnp.stack([tok0, tok1]).astype(jnp.int32)
    pad = (-tok.shape[1]) % 16
    if pad:
        tok = jnp.concatenate([tok, tok[:, :pad]], axis=1)
    kidx = _sc_compact(tok, L).reshape(2, 1, _M)

    xs = _self_layer(x_raw, pe, kidx,
                     Wq[0], Wk[0], Wv[0], Wo[0], W1[0], W2[0])
    xc = _cross_layer(xs, wmask, Wq[1], Wk[1], Wv[1], Wo[1], W1[1], W2[1],
                      shifts)
    return xc[0][None], xc[1][None]


# speedup vs baseline: 1.2718x; 1.1722x over previous
import functools
import math

import numpy as np
import jax
import jax.numpy as jnp
from jax import lax
from jax.experimental import pallas as pl
from jax.experimental.pallas import tpu as pltpu
from jax.experimental.pallas import tpu_sc as plsc

NHEAD = 8
WSZ = 5
_F32 = jnp.float32


def _sine_pos_encoding_np(C, H, W):
    pe = np.zeros((C, H, W), dtype=np.float32)
    yy = np.tile(np.arange(H, dtype=np.float32)[:, None], (1, W))
    xx = np.tile(np.arange(W, dtype=np.float32)[None, :], (H, 1))
    div = np.exp(np.arange(0, C // 2, 2).astype(np.float32) * (-math.log(10000.0) / (C // 2)))
    d = div[:, None, None]
    pe[0::4] = np.sin(xx[None] * d)
    pe[1::4] = np.cos(xx[None] * d)
    pe[2::4] = np.sin(yy[None] * d)
    pe[3::4] = np.cos(yy[None] * d)
    return pe


def _window_valid_np(hh, ww, wsz):
    L = hh * ww
    r = np.arange(L) // ww
    c = np.arange(L) % ww
    off = np.arange(wsz) - wsz // 2
    dr = np.repeat(off, wsz)
    dc = np.tile(off, wsz)
    rr = r[:, None] + dr[None, :]
    cc = c[:, None] + dc[None, :]
    return ((rr >= 0) & (rr < hh) & (cc >= 0) & (cc < ww)).astype(np.float32)


_M = 1536


def _sc_compact(tok_pad, L):
    npad = tok_pad.shape[1]
    mesh = plsc.VectorSubcoreMesh(core_axis_name="c", subcore_axis_name="s")

    @functools.partial(
        pl.kernel,
        out_type=jax.ShapeDtypeStruct((2, _M), jnp.int32),
        mesh=mesh,
        scratch_types=[
            pltpu.VMEM((npad,), jnp.int32),
            pltpu.VMEM((L,), _F32),
            pltpu.VMEM((_M,), jnp.int32),
        ],
        compiler_params=pltpu.CompilerParams(needs_layout_passes=False),
    )
    def build(tok_hbm, kidx_hbm, tok_v, mask_v, kidx_v):
        wid = lax.axis_index("s") * 2 + lax.axis_index("c")

        @pl.when(wid < 2)
        def _():
            pltpu.sync_copy(tok_hbm.at[wid], tok_v)
            zeros = jnp.zeros((16,), _F32)
            ones = jnp.ones((16,), _F32)
            ineg = jnp.full((16,), -1, jnp.int32)
            lanes = lax.iota(jnp.int32, 16)

            def init_body(i, carry):
                mask_v[pl.ds(i * 16, 16)] = zeros
                return carry

            lax.fori_loop(0, L // 16, init_body, 0)

            def scat_body(i, carry):
                idx = tok_v[pl.ds(i * 16, 16)]
                plsc.store_scatter(mask_v, [idx], ones)
                return carry

            lax.fori_loop(0, npad // 16, scat_body, 0)

            def kinit_body(i, carry):
                kidx_v[pl.ds(i * 16, 16)] = ineg
                return carry

            lax.fori_loop(0, _M // 16, kinit_body, 0)

            def compact_body(i, base):
                m = mask_v[pl.ds(i * 16, 16)]
                hit = m > 0.5
                c = plsc.cumsum(m)
                pos = (base + c - 1.0).astype(jnp.int32)
                pos = jnp.where(hit, pos, 0)
                tid = lanes + i * 16
                plsc.store_scatter(kidx_v, [pos], tid, mask=hit)
                return base + jnp.sum(m)

            lax.fori_loop(0, L // 16, compact_body, jnp.float32(0.0))
            pltpu.sync_copy(kidx_v, kidx_hbm.at[wid])

    return build(tok_pad)


def _ffn_block(o, w1, w2):
    mu = jnp.mean(o, axis=-1, keepdims=True)
    var = jnp.mean((o - mu) * (o - mu), axis=-1, keepdims=True)
    ln = (o - mu) / jnp.sqrt(var + 1e-6)
    h = jnp.maximum(jnp.dot(ln, w1, preferred_element_type=_F32), 0.0)
    return o + jnp.dot(h, w2, preferred_element_type=_F32)


def _self_body(x_ref, pe_ref, ki_ref, wq_ref, wk_ref, wv_ref, wo_ref,
               w1_ref, w2_ref, o_ref, q_s, k_s, v_s, *, L, C, hd, tq):
    x = x_ref[0] + pe_ref[...]
    scale = 1.0 / math.sqrt(hd)
    q_s[...] = (jnp.dot(x, wq_ref[...], preferred_element_type=_F32)
                * scale).astype(jnp.bfloat16)
    kcol = jnp.transpose(ki_ref[0])
    oh = (kcol == lax.broadcasted_iota(jnp.int32, (_M, L), 1)
          ).astype(jnp.bfloat16)
    xg = jnp.dot(oh, x.astype(jnp.bfloat16), preferred_element_type=_F32)
    k_s[...] = jnp.dot(xg, wk_ref[...],
                       preferred_element_type=_F32).astype(jnp.bfloat16)
    v_s[...] = jnp.dot(xg, wv_ref[...],
                       preferred_element_type=_F32).astype(jnp.bfloat16)
    npad = jnp.sum((ki_ref[0] == -1).astype(_F32), axis=-1, keepdims=True)
    wo = wo_ref[...]
    w1 = w1_ref[...]
    w2 = w2_ref[...]
    for t in range(L // tq):
        sl = slice(t * tq, (t + 1) * tq)
        msg_parts = []
        for h in range(NHEAD):
            hs = slice(h * hd, (h + 1) * hd)
            lg = lax.dot_general(q_s[sl, hs], k_s[:, hs],
                                 (((1,), (1,)), ((), ())),
                                 preferred_element_type=_F32)
            p32 = jnp.exp(lg)
            s = jnp.sum(p32, axis=-1, keepdims=True) - npad
            pv = jnp.dot(p32.astype(jnp.bfloat16), v_s[:, hs],
                         preferred_element_type=_F32)
            msg_parts.append(pv * (1.0 / s))
        msg = jnp.concatenate(msg_parts, axis=-1)
        o = (x_ref[0, sl, :] + pe_ref[sl, :]
             + jnp.dot(msg, wo, preferred_element_type=_F32))
        o_ref[0, sl, :] = _ffn_block(o, w1, w2)


def _self_layer(x_raw, pe, kidx, wq, wk, wv, wo, w1, w2):
    _, L, C = x_raw.shape
    hd = C // NHEAD
    body = functools.partial(_self_body, L=L, C=C, hd=hd, tq=1024)
    full2 = lambda i: (0, 0)
    return pl.pallas_call(
        body,
        grid=(2,),
        in_specs=[
            pl.BlockSpec((1, L, C), lambda i: (i, 0, 0)),
            pl.BlockSpec((L, C), full2),
            pl.BlockSpec((1, 1, _M), lambda i: (i, 0, 0)),
            pl.BlockSpec((C, C), full2),
            pl.BlockSpec((C, C), full2),
            pl.BlockSpec((C, C), full2),
            pl.BlockSpec((C, C), full2),
            pl.BlockSpec((C, 2 * C), full2),
            pl.BlockSpec((2 * C, C), full2),
        ],
        out_specs=pl.BlockSpec((1, L, C), lambda i: (i, 0, 0)),
        out_shape=jax.ShapeDtypeStruct((2, L, C), _F32),
        scratch_shapes=[
            pltpu.VMEM((L, C), jnp.bfloat16),
            pltpu.VMEM((_M, C), jnp.bfloat16),
            pltpu.VMEM((_M, C), jnp.bfloat16),
        ],
    )(x_raw, pe, kidx, wq, wk, wv, wo, w1, w2)


_HALO = 144


def _cross_body(xq_ref, xkv_ref, wm_ref, wq_ref, wk_ref, wv_ref, wo_ref,
                w1_ref, w2_ref, o_ref, kpad_ref, vpad_ref,
                *, L, C, hd, ww, tq):
    xq = xq_ref[0]
    xkv = xkv_ref[0]
    bf16 = jnp.bfloat16
    kv = jnp.dot(xkv, wk_ref[...], preferred_element_type=_F32).astype(bf16)
    vv = jnp.dot(xkv, wv_ref[...], preferred_element_type=_F32).astype(bf16)
    ztop = jnp.zeros((_HALO + 8, C), bf16)
    zbot = jnp.zeros((_HALO + 8, C), bf16)
    for dci, dc in enumerate(range(-2, 3)):
        kpad_ref[dci, :_HALO + 8, :] = ztop
        kpad_ref[dci, _HALO - 8 + L:, :] = zbot
        kpad_ref[dci, _HALO - dc:_HALO - dc + L, :] = kv
        vpad_ref[dci, :_HALO + 8, :] = ztop
        vpad_ref[dci, _HALO - 8 + L:, :] = zbot
        vpad_ref[dci, _HALO - dc:_HALO - dc + L, :] = vv
    scale = 1.0 / math.sqrt(hd)
    di = lax.broadcasted_iota(jnp.int32, (C, NHEAD), 0)
    hi = lax.broadcasted_iota(jnp.int32, (C, NHEAD), 1)
    e = (di // hd == hi).astype(bf16)
    et32 = (lax.broadcasted_iota(jnp.int32, (NHEAD, C), 1) // hd ==
            lax.broadcasted_iota(jnp.int32, (NHEAD, C), 0)).astype(_F32)
    et = et32.astype(bf16)
    wq = wq_ref[...]
    wo = wo_ref[...]
    w1 = w1_ref[...]
    w2 = w2_ref[...]

    for t in range(L // tq):
        sl = slice(t * tq, (t + 1) * tq)
        xq_t = xq[sl]
        q_t = (jnp.dot(xq_t, wq, preferred_element_type=_F32)
               * scale).astype(bf16)
        lgts = []
        for w in range(WSZ * WSZ):
            dri, dci = w // WSZ, w % WSZ
            off = _HALO + t * tq + (dri - 2) * ww
            ks = kpad_ref[dci, off:off + tq, :]
            lg = jnp.dot(q_t * ks, e, preferred_element_type=_F32)
            valid = wm_ref[sl, w:w + 1]
            lgts.append(jnp.where(valid > 0.5, lg, -1e9))
        mx = lgts[0]
        for lg in lgts[1:]:
            mx = jnp.maximum(mx, lg)
        ps = [jnp.exp(lg - mx) for lg in lgts]
        denom = ps[0]
        for p in ps[1:]:
            denom = denom + p
        msg = jnp.zeros((tq, C), _F32)
        for w in range(WSZ * WSZ):
            dri, dci = w // WSZ, w % WSZ
            off = _HALO + t * tq + (dri - 2) * ww
            vs = vpad_ref[dci, off:off + tq, :]
            pexp = jnp.dot(ps[w].astype(bf16), et,
                           preferred_element_type=_F32).astype(bf16)
            msg = msg + (pexp * vs).astype(_F32)
        msg = msg * jnp.dot(1.0 / denom, et32, preferred_element_type=_F32)
        o = xq_t + jnp.dot(msg, wo, preferred_element_type=_F32)
        o_ref[0, sl, :] = _ffn_block(o, w1, w2)


def _cross_layer(xq, wmask, wq, wk, wv, wo, w1, w2, ww):
    _, L, C = xq.shape
    hd = C // NHEAD
    body = functools.partial(_cross_body, L=L, C=C, hd=hd, ww=ww,
                             tq=1024)
    full2 = lambda i: (0, 0)
    return pl.pallas_call(
        body,
        grid=(2,),
        in_specs=[
            pl.BlockSpec((1, L, C), lambda i: (i, 0, 0)),
            pl.BlockSpec((1, L, C), lambda i: (1 - i, 0, 0)),
            pl.BlockSpec((L, WSZ * WSZ), full2),
            pl.BlockSpec((C, C), full2),
            pl.BlockSpec((C, C), full2),
            pl.BlockSpec((C, C), full2),
            pl.BlockSpec((C, C), full2),
            pl.BlockSpec((C, 2 * C), full2),
            pl.BlockSpec((2 * C, C), full2),
        ],
        out_specs=pl.BlockSpec((1, L, C), lambda i: (i, 0, 0)),
        out_shape=jax.ShapeDtypeStruct((2, L, C), _F32),
        scratch_shapes=[
            pltpu.VMEM((5, 2 * _HALO + L, C), jnp.bfloat16),
            pltpu.VMEM((5, 2 * _HALO + L, C), jnp.bfloat16),
        ],
    )(xq, xq, wmask, wq, wk, wv, wo, w1, w2)


def kernel(cnn_desc0, cnn_desc1, mkpts0_c, mkpts1_c, m_bids, image0, image1,
           Wq, Wk, Wv, Wo, W1, W2):
    B, C, hh, ww = cnn_desc0.shape
    L = hh * ww
    scale = image0.shape[2] // hh
    hd = C // NHEAD

    pe = jnp.asarray(_sine_pos_encoding_np(C, hh, ww).reshape(C, L).T)
    wmask = jnp.asarray(_window_valid_np(hh, ww, WSZ))

    x_raw = jnp.stack([
        cnn_desc0.reshape(C, L).T,
        cnn_desc1.reshape(C, L).T,
    ])

    tok0 = (mkpts0_c[:, 1] // scale) * ww + (mkpts0_c[:, 0] // scale)
    tok1 = (mkpts1_c[:, 1] // scale) * ww + (mkpts1_c[:, 0] // scale)
    tok = jnp.stack([tok0, tok1]).astype(jnp.int32)
    pad = (-tok.shape[1]) % 16
    if pad:
        tok = jnp.concatenate([tok, tok[:, :pad]], axis=1)
    kidx = _sc_compact(tok, L).reshape(2, 1, _M)

    xs = _self_layer(x_raw, pe, kidx,
                     Wq[0], Wk[0], Wv[0], Wo[0], W1[0], W2[0])
    xc = _cross_layer(xs, wmask, Wq[1], Wk[1], Wv[1], Wo[1], W1[1], W2[1],
                      ww)
    return xc[0][None], xc[1][None]
